# Initial kernel scaffold; baseline (speedup 1.0000x reference)
import jax
import jax.numpy as jnp
from jax.experimental import pallas as pl


def kernel(images, enc_params, init_params, dec_params):
    raise NotImplementedError("write your pallas kernel here")


# speedup vs baseline: 1.1309x; 1.1309x over previous
import math
import jax
import jax.numpy as jnp
from jax.experimental import pallas as pl

_GROWTH = 24
_N_DENSE = 16
_WORD_NUM = 111
_HIDDEN = 256
_ATT_DIM = 512
_OUT_CH = 684
_MAX_STEPS = 200
_KS = 11
_PAD = _KS // 2

_HP = jax.lax.Precision.HIGHEST


def _conv2d(x, w, stride=1, padding=0, bias=None):
    out = jax.lax.conv_general_dilated(
        x, w, (stride, stride), [(padding, padding), (padding, padding)],
        dimension_numbers=('NCHW', 'OIHW', 'NCHW'))
    if bias is not None:
        out = out + bias[None, :, None, None]
    return out


def _bn(x, s, b):
    return x * s[None, :, None, None] + b[None, :, None, None]


def _maxpool2(x):
    return jax.lax.reduce_window(x, -jnp.inf, jax.lax.max, (1, 1, 2, 2), (1, 1, 2, 2), 'VALID')


def _avgpool2(x):
    return jax.lax.reduce_window(x, 0.0, jax.lax.add, (1, 1, 2, 2), (1, 1, 2, 2), 'VALID') * 0.25


def _block(x, layers):
    for p in layers:
        o = jax.nn.relu(_bn(_conv2d(x, p['c1']), p['s1'], p['b1']))
        o = jax.nn.relu(_bn(_conv2d(o, p['c2'], padding=1), p['s2'], p['b2']))
        x = jnp.concatenate([x, o], axis=1)
    return x


def _counting(x, p):
    k = p['trans_w'].shape[-1]
    t = _bn(_conv2d(x, p['trans_w'], padding=k // 2), p['bn_s'], p['bn_b'])
    y = t.mean(axis=(2, 3))
    y = jax.nn.relu(y @ p['fc1_w'].T + p['fc1_b'])
    y = jax.nn.sigmoid(y @ p['fc2_w'].T + p['fc2_b'])
    t = t * y[:, :, None, None]
    pred = jax.nn.sigmoid(_conv2d(t, p['pred_w']))
    return pred.sum(axis=(2, 3))


def _pos_embed_sine(B, H, W, num_feats=256, temperature=10000.0):
    ones = jnp.ones((B, H, W), jnp.float32)
    y = jnp.cumsum(ones, axis=1)
    x = jnp.cumsum(ones, axis=2)
    scale = 2.0 * math.pi
    eps = 1e-6
    y = y / (y[:, -1:, :] + eps) * scale
    x = x / (x[:, :, -1:] + eps) * scale
    dim_t = jnp.arange(num_feats, dtype=jnp.float32)
    dim_t = temperature ** (2.0 * jnp.floor(dim_t / 2.0) / num_feats)
    px = x[..., None] / dim_t
    py = y[..., None] / dim_t
    px = jnp.stack([jnp.sin(px[..., 0::2]), jnp.cos(px[..., 1::2])], axis=-1).reshape(B, H, W, num_feats)
    py = jnp.stack([jnp.sin(py[..., 0::2]), jnp.cos(py[..., 1::2])], axis=-1).reshape(B, H, W, num_feats)
    return jnp.concatenate([py, px], axis=-1).transpose(0, 3, 1, 2)


def _decoder_body(cnnT_ref, trans_ref, cctx_ref, h0_ref,
                  emb_ref, w_ihT_ref, b_ih_ref, w_hhT_ref, b_hh_ref,
                  hid_wT_ref, hid_b_ref, wc_ref, alpha_wT_ref, alpha_b_ref,
                  state_wT_ref, state_b_ref, emb_wT_ref, emb_b_ref,
                  ctx_wT_ref, ctx_b_ref, conv_wT_ref, conv_b_ref,
                  logits_out, alpha_out, word_out, valid_out,
                  H, W):
    L = H * W
    HP = (H + 2 * _PAD, W + 2 * _PAD)

    cnnT = cnnT_ref[:, :]
    trans = trans_ref[:, :]
    cctx = cctx_ref[:, :]
    emb_tab = emb_ref[:, :]
    w_ihT = w_ihT_ref[:, :]
    b_ih = b_ih_ref[:, :]
    w_hhT = w_hhT_ref[:, :]
    b_hh = b_hh_ref[:, :]
    hid_wT = hid_wT_ref[:, :]
    hid_b = hid_b_ref[:, :]
    wc = wc_ref[:, :]
    alpha_wT = alpha_wT_ref[:, :]
    alpha_b = alpha_b_ref[:, :]
    state_wT = state_wT_ref[:, :]
    state_b = state_b_ref[:, :]
    emb_wT = emb_wT_ref[:, :]
    emb_b = emb_b_ref[:, :]
    ctx_wT = ctx_wT_ref[:, :]
    ctx_b = ctx_b_ref[:, :]
    conv_wT = conv_wT_ref[:, :]
    conv_b = conv_b_ref[:, :]

    word_iota = jax.lax.broadcasted_iota(jnp.int32, (1, _WORD_NUM), 1)

    def step(t, carry):
        word, hidden, padded, done = carry
        onehot = (word_iota == word).astype(jnp.float32)
        emb = jnp.dot(onehot, emb_tab, precision=_HP)
        gi = jnp.dot(emb, w_ihT, precision=_HP) + b_ih
        gh = jnp.dot(hidden, w_hhT, precision=_HP) + b_hh
        r = jax.nn.sigmoid(gi[:, :_HIDDEN] + gh[:, :_HIDDEN])
        z = jax.nn.sigmoid(gi[:, _HIDDEN:2 * _HIDDEN] + gh[:, _HIDDEN:2 * _HIDDEN])
        n = jnp.tanh(gi[:, 2 * _HIDDEN:] + r * gh[:, 2 * _HIDDEN:])
        hidden = (1.0 - z) * n + z * hidden
        query = jnp.dot(hidden, hid_wT, precision=_HP) + hid_b
        views = [padded[ky:ky + H, kx:kx + W]
                 for ky in range(_KS) for kx in range(_KS)]
        views += [jnp.zeros((H, W), jnp.float32)] * (128 - _KS * _KS)
        patches = jnp.stack(views, axis=0).reshape(128, L)
        covp = jax.lax.dot_general(patches, wc, (((0,), (0,)), ((), ())),
                                   precision=_HP)
        score = jnp.tanh(query + covp + trans)
        energy = jnp.dot(score, alpha_wT, precision=_HP) + alpha_b
        energy = energy - jnp.max(energy)
        e = jnp.exp(energy)
        alpha = e / (jnp.sum(e) + 1e-10)
        a_hw = alpha.reshape(H, W)
        z_lr = jnp.zeros((H, _PAD), jnp.float32)
        z_tb = jnp.zeros((_PAD, W + 2 * _PAD), jnp.float32)
        mid = jnp.concatenate([z_lr, a_hw, z_lr], axis=1)
        padded = padded + jnp.concatenate([z_tb, mid, z_tb], axis=0)
        ctx = jax.lax.dot_general(alpha, cnnT, (((0,), (0,)), ((), ())),
                                  precision=_HP)
        out_state = (jnp.dot(hidden, state_wT, precision=_HP) + state_b
                     + jnp.dot(emb, emb_wT, precision=_HP) + emb_b
                     + jnp.dot(ctx, ctx_wT, precision=_HP) + ctx_b
                     + cctx)
        logits = jnp.dot(out_state, conv_wT, precision=_HP) + conv_b
        new_word = jnp.argmax(logits, axis=1).astype(jnp.int32).reshape(1, 1)
        is_eos = (new_word == 0).astype(jnp.float32)
        valid = (1.0 - done) * (1.0 - is_eos)
        done = jnp.maximum(done, is_eos)
        logits_out[pl.ds(t, 1), :] = logits * valid
        alpha_out[pl.ds(t, 1), :] = alpha.reshape(1, L) * valid
        word_out[pl.ds(t, 1), :] = new_word
        valid_out[pl.ds(t, 1), :] = valid
        return new_word, hidden, padded, done

    word0 = jnp.ones((1, 1), jnp.int32)
    padded0 = jnp.zeros(HP, jnp.float32)
    done0 = jnp.zeros((1, 1), jnp.float32)
    jax.lax.fori_loop(0, _MAX_STEPS, step, (word0, h0_ref[:, :], padded0, done0))


def kernel(images, enc_params, init_params, dec_params):
    x = _conv2d(images, enc_params['conv1_w'], stride=2, padding=3)
    x = _maxpool2(jax.nn.relu(x))
    x = _block(x, enc_params['block1'])
    t = enc_params['trans1']
    x = _avgpool2(jax.nn.relu(_bn(_conv2d(x, t['w']), t['s'], t['b'])))
    x = _block(x, enc_params['block2'])
    t = enc_params['trans2']
    x = _avgpool2(jax.nn.relu(_bn(_conv2d(x, t['w']), t['s'], t['b'])))
    cnn = _block(x, enc_params['block3'])
    counting_preds = 0.5 * (_counting(cnn, enc_params['count3'])
                            + _counting(cnn, enc_params['count5']))

    B, C, H, W = cnn.shape
    L = H * W

    trans = _conv2d(cnn, init_params['feat_conv_w'], bias=init_params['feat_conv_b'])
    trans = trans + _pos_embed_sine(B, H, W)
    counting_ctx = counting_preds @ init_params['count_ctx_w'].T + init_params['count_ctx_b']
    avg = cnn.mean(axis=(2, 3))
    hidden0 = jnp.tanh(avg @ init_params['init_w'].T + init_params['init_b'])

    dp = dec_params
    cnnT = cnn.reshape(C, L).T
    trans_flat = trans.reshape(_ATT_DIM, L).T
    acw_flat = dp['att_conv_w'].reshape(_ATT_DIM, _KS * _KS)
    wc = jnp.dot(acw_flat.T, dp['att_w'].T, precision=_HP)
    wc = jnp.pad(wc, ((0, 128 - _KS * _KS), (0, 0)))

    row = lambda v: v.reshape(1, -1).astype(jnp.float32)
    alpha_wT = dp['alpha_w'].T
    alpha_b = dp['alpha_b'].reshape(1, 1)
    operands = [
        cnnT, trans_flat, counting_ctx, hidden0,
        dp['emb'], dp['w_ih'].T, row(dp['b_ih']), dp['w_hh'].T, row(dp['b_hh']),
        dp['hid_w'].T, row(dp['hid_b']), wc, alpha_wT, alpha_b,
        dp['state_w'].T, row(dp['state_b']), dp['emb_w'].T, row(dp['emb_b']),
        dp['ctx_w'].T, row(dp['ctx_b']), dp['conv_w'].T, row(dp['conv_b']),
    ]

    out_shape = [
        jax.ShapeDtypeStruct((_MAX_STEPS, _WORD_NUM), jnp.float32),
        jax.ShapeDtypeStruct((_MAX_STEPS, L), jnp.float32),
        jax.ShapeDtypeStruct((_MAX_STEPS, 1), jnp.int32),
        jax.ShapeDtypeStruct((_MAX_STEPS, 1), jnp.float32),
    ]

    import functools
    logits_seq, alpha_seq, word_seq, valid_seq = pl.pallas_call(
        functools.partial(_decoder_body, H=H, W=W),
        out_shape=out_shape,
    )(*operands)

    logits_seq = logits_seq[:, None, :]
    alpha_seq = alpha_seq.reshape(_MAX_STEPS, 1, H, W)
    word_seq = word_seq
    valid_seq = valid_seq > 0.5
    return logits_seq, alpha_seq, word_seq, valid_seq
